# Initial kernel scaffold; baseline (speedup 1.0000x reference)
import jax
import jax.numpy as jnp
from jax.experimental import pallas as pl


def kernel(x, edge_index, weight, W1, b1, W2, b2, W3, b3):
    raise NotImplementedError("write your pallas kernel here")


# speedup vs baseline: 2.2375x; 2.2375x over previous
import functools

import jax
import jax.numpy as jnp
from jax import lax
from jax.experimental import pallas as pl
from jax.experimental.pallas import tpu as pltpu
from jax.experimental.pallas import tpu_sc as plsc

N = 10000
E = 320000
D = 128
K = 5
N_CLASSES = 10

NC = 2
NS = 16
NW = NC * NS

NPAD = 10240
RPT = NPAD // NS
EPAD = 327680
EPT = EPAD // NW
CH = 128
NCH = EPT // CH

_mesh = plsc.VectorSubcoreMesh(core_axis_name="c", subcore_axis_name="s")
_sc_params = pltpu.CompilerParams(needs_layout_passes=False)


@functools.partial(
    pl.kernel,
    mesh=_mesh,
    compiler_params=_sc_params,
    out_type=jax.ShapeDtypeStruct((NC, NPAD, D), jnp.float32),
    scratch_types=[
        pltpu.VMEM((CH,), jnp.int32),
        pltpu.VMEM((CH,), jnp.int32),
        pltpu.VMEM((CH,), jnp.float32),
        pltpu.VMEM((CH, D), jnp.float32),
        pltpu.VMEM_SHARED((NPAD, D), jnp.float32),
        pltpu.SemaphoreType.DMA,
    ],
)
def _hop(h_hbm, src_hbm, dst_hbm, norm_hbm, out_hbm,
         src_v, dst_v, norm_v, rows_v, acc_sh, sem):
    cid = lax.axis_index("c")
    sid = lax.axis_index("s")
    wid = sid * NC + cid

    def _zero_row(r, _):
        for j in range(D // 16):
            rows_v[r, pl.ds(j * 16, 16)] = jnp.zeros((16,), jnp.float32)
        return 0
    lax.fori_loop(0, CH, _zero_row, 0)
    for r0 in range(0, RPT, CH):
        sz = min(CH, RPT - r0)
        pltpu.sync_copy(rows_v.at[pl.ds(0, sz)],
                        acc_sh.at[pl.ds(sid * RPT + r0, sz)])
    plsc.subcore_barrier()

    def _chunk(k, _):
        base = pl.multiple_of(wid * EPT + k * CH, CH)
        pltpu.sync_copy(src_hbm.at[pl.ds(base, CH)], src_v)
        pltpu.sync_copy(dst_hbm.at[pl.ds(base, CH)], dst_v)
        pltpu.sync_copy(norm_hbm.at[pl.ds(base, CH)], norm_v)
        pltpu.async_copy(h_hbm.at[src_v], rows_v, sem).wait()

        def _scale(e, _):
            lanes = jnp.broadcast_to(e.astype(jnp.int32), (16,))
            nb = plsc.load_gather(norm_v, [lanes])
            for j in range(D // 16):
                sl = pl.ds(j * 16, 16)
                rows_v[e, sl] = rows_v[e, sl] * nb
            return 0
        lax.fori_loop(0, CH, _scale, 0)

        pltpu.sync_copy(rows_v, acc_sh.at[dst_v], add=True)
        return 0
    lax.fori_loop(0, NCH, _chunk, 0)

    plsc.subcore_barrier()
    pltpu.sync_copy(acc_sh.at[pl.ds(sid * RPT, RPT)],
                    out_hbm.at[cid, pl.ds(sid * RPT, RPT)])


@functools.partial(
    pl.kernel,
    mesh=_mesh,
    compiler_params=_sc_params,
    out_type=jax.ShapeDtypeStruct((EPAD,), jnp.float32),
    scratch_types=[
        pltpu.VMEM((NPAD,), jnp.float32),
        pltpu.VMEM((CH,), jnp.int32),
        pltpu.VMEM((CH,), jnp.int32),
        pltpu.VMEM((CH,), jnp.float32),
        pltpu.VMEM((CH,), jnp.float32),
    ],
)
def _edge_norm(dis_hbm, src_hbm, dst_hbm, w_hbm, out_hbm,
               dis_v, src_v, dst_v, w_v, nrm_v):
    cid = lax.axis_index("c")
    sid = lax.axis_index("s")
    wid = sid * NC + cid
    pltpu.sync_copy(dis_hbm, dis_v)

    def _chunk(k, _):
        base = pl.multiple_of(wid * EPT + k * CH, CH)
        pltpu.sync_copy(src_hbm.at[pl.ds(base, CH)], src_v)
        pltpu.sync_copy(dst_hbm.at[pl.ds(base, CH)], dst_v)
        pltpu.sync_copy(w_hbm.at[pl.ds(base, CH)], w_v)
        for t in range(CH // 16):
            sl = pl.ds(t * 16, 16)
            a = plsc.load_gather(dis_v, [src_v[sl]])
            b = plsc.load_gather(dis_v, [dst_v[sl]])
            nrm_v[sl] = a * w_v[sl] * b
        pltpu.sync_copy(nrm_v, out_hbm.at[pl.ds(base, CH)])
        return 0
    lax.fori_loop(0, NCH, _chunk, 0)


_BM = NPAD // 16


def _merge_body(p_ref, o_ref):
    o_ref[...] = p_ref[0] + p_ref[1]


def _merge_rsqrt_body(p_ref, o_ref):
    s = p_ref[0] + p_ref[1]
    safe = jnp.where(s > 0, s, 1.0)
    o_ref[...] = jnp.where(s > 0, lax.rsqrt(safe), 0.0)


def _merge(parts, rsqrt=False):
    body = _merge_rsqrt_body if rsqrt else _merge_body
    return pl.pallas_call(
        body,
        grid=(16,),
        in_specs=[pl.BlockSpec((NC, _BM, D), lambda i: (0, i, 0))],
        out_specs=pl.BlockSpec((_BM, D), lambda i: (i, 0)),
        out_shape=jax.ShapeDtypeStruct((NPAD, D), jnp.float32),
    )(parts)


def _mm_body(act, x0, x1, x2, x3, x4, x5, w_ref, b_ref, o_ref):
    acc = jnp.broadcast_to(b_ref[0, :], (_BM, D)).astype(jnp.float32)
    for i, xr in enumerate((x0, x1, x2, x3, x4, x5)):
        acc = acc + jnp.dot(xr[...], w_ref[i * D:(i + 1) * D, :],
                            preferred_element_type=jnp.float32)
    if act:
        acc = jnp.where(acc > 0, acc, jnp.exp(jnp.minimum(acc, 0.0)) - 1.0)
    o_ref[...] = acc


def _tag_matmul(xs, W, b, act):
    xspec = pl.BlockSpec((_BM, D), lambda i: (i, 0))
    return pl.pallas_call(
        functools.partial(_mm_body, act),
        grid=(16,),
        in_specs=[xspec] * 6 + [
            pl.BlockSpec((6 * D, D), lambda i: (0, 0)),
            pl.BlockSpec((1, D), lambda i: (0, 0)),
        ],
        out_specs=pl.BlockSpec((_BM, D), lambda i: (i, 0)),
        out_shape=jax.ShapeDtypeStruct((NPAD, D), jnp.float32),
    )(*xs, W, b)


def kernel(x, edge_index, weight, W1, b1, W2, b2, W3, b3):
    src = edge_index[0].astype(jnp.int32)
    dst = edge_index[1].astype(jnp.int32)
    src_p = jnp.pad(src, (0, EPAD - E))
    dst_p = jnp.pad(dst, (0, EPAD - E))
    w_p = jnp.pad(weight, (0, EPAD - E))

    ones_h = jnp.ones((NPAD, D), jnp.float32)
    deg_parts = _hop(ones_h, src_p, dst_p, w_p)
    dis2d = _merge(deg_parts, rsqrt=True)
    dis = dis2d[:, 0]
    norm = _edge_norm(dis, src_p, dst_p, w_p)

    W3p = jnp.pad(W3, ((0, 0), (0, D - N_CLASSES)))
    b3p = jnp.pad(b3, (0, D - N_CLASSES))

    h = jnp.pad(x, ((0, NPAD - N), (0, 0)))
    for W, b, act in ((W1, b1, True), (W2, b2, True), (W3p, b3p, False)):
        xs = [h]
        for _ in range(K):
            parts = _hop(xs[-1], src_p, dst_p, norm)
            xs.append(_merge(parts))
        h = _tag_matmul(xs, W, b.reshape(1, D), act)
    return h[:N, :N_CLASSES]
